# Initial kernel scaffold; baseline (speedup 1.0000x reference)
import jax
import jax.numpy as jnp
from jax.experimental import pallas as pl


def kernel(r, e0, e1, V, R):
    raise NotImplementedError("write your pallas kernel here")


# speedup vs baseline: 2.7523x; 2.7523x over previous
import functools

import jax
import jax.numpy as jnp
from jax import lax
from jax.experimental import pallas as pl
from jax.experimental.pallas import tpu as pltpu
from jax.experimental.pallas import tpu_sc as plsc

B, L, D = 4096, 50, 64
LANES = 16
KD = D // LANES

try:
    _info = plsc.get_sparse_core_info()
    NC, NS = int(_info.num_cores), int(_info.num_subcores)
except Exception:
    NC, NS = 2, 16
NW = NC * NS
RPW = B // NW
CB = 2
E = CB * L
NCHUNK = RPW // CB


def _body(r_h, e0_h, e1_h, v_h, t_h, out_h,
          idx_r, idx_0, idx_1,
          bR0, bR1, b00, b01, b10, b11,
          out_v, sem0, sem1):
    wid = lax.axis_index("s") * NC + lax.axis_index("c")

    pltpu.sync_copy(r_h.at[wid], idx_r)
    pltpu.sync_copy(e0_h.at[wid], idx_0)
    pltpu.sync_copy(e1_h.at[wid], idx_1)

    bufs = ((bR0, b00, b10, sem0), (bR1, b01, b11, sem1))

    def issue(c, slot):
        bR, b0, b1, sem = slot
        pltpu.async_copy(t_h.at[idx_r.at[c]], bR, sem)
        pltpu.async_copy(v_h.at[idx_0.at[c]], b0, sem)
        pltpu.async_copy(v_h.at[idx_1.at[c]], b1, sem)

    def drain(c, slot):
        bR, b0, b1, sem = slot
        pltpu.make_async_copy(t_h.at[idx_r.at[c]], bR, sem).wait()
        pltpu.make_async_copy(v_h.at[idx_0.at[c]], b0, sem).wait()
        pltpu.make_async_copy(v_h.at[idx_1.at[c]], b1, sem).wait()

    def compute(c, slot):
        bR, b0, b1, _ = slot
        zero = jnp.zeros((LANES,), jnp.float32)

        def jbody(j, accs):
            new = []
            for row in range(CB):
                base = row * L + j
                for k in range(KD):
                    sl = pl.ds(LANES * k, LANES)
                    new.append(accs[row * KD + k]
                               + bR[base, sl] * b0[base, sl] * b1[base, sl])
            return tuple(new)

        accs = lax.fori_loop(0, L, jbody, (zero,) * (CB * KD))
        for row in range(CB):
            for k in range(KD):
                out_v[c * CB + row, pl.ds(LANES * k, LANES)] = accs[row * KD + k]

    issue(0, bufs[0])
    issue(1, bufs[1])

    def outer(i, _):
        for b in range(2):
            c = i * 2 + b
            drain(c, bufs[b])
            compute(c, bufs[b])
            issue(c + 2, bufs[b])
        return 0

    lax.fori_loop(0, (NCHUNK - 2) // 2, outer, 0)
    for b in range(2):
        c = NCHUNK - 2 + b
        drain(c, bufs[b])
        compute(c, bufs[b])

    pltpu.sync_copy(out_v, out_h.at[pl.ds(wid * RPW, RPW)])


@functools.partial(jax.jit, static_argnames=())
def kernel(r, e0, e1, V, R):
    mesh = plsc.VectorSubcoreMesh(core_axis_name="c", subcore_axis_name="s")
    kfn = pl.kernel(
        _body,
        out_type=jax.ShapeDtypeStruct((B, D), jnp.float32),
        mesh=mesh,
        compiler_params=pltpu.CompilerParams(use_tc_tiling_on_sc=False),
        scratch_types=[
            pltpu.VMEM((NCHUNK, E), jnp.int32),
            pltpu.VMEM((NCHUNK, E), jnp.int32),
            pltpu.VMEM((NCHUNK, E), jnp.int32),
            pltpu.VMEM((E, D), jnp.float32),
            pltpu.VMEM((E, D), jnp.float32),
            pltpu.VMEM((E, D), jnp.float32),
            pltpu.VMEM((E, D), jnp.float32),
            pltpu.VMEM((E, D), jnp.float32),
            pltpu.VMEM((E, D), jnp.float32),
            pltpu.VMEM((RPW, D), jnp.float32),
            pltpu.SemaphoreType.DMA,
            pltpu.SemaphoreType.DMA,
        ],
    )
    r3 = r.astype(jnp.int32).reshape(NW, NCHUNK, E)
    e03 = e0.astype(jnp.int32).reshape(NW, NCHUNK, E)
    e13 = e1.astype(jnp.int32).reshape(NW, NCHUNK, E)
    return kfn(r3, e03, e13, V, R)
